# Initial kernel scaffold; baseline (speedup 1.0000x reference)
import jax
import jax.numpy as jnp
from jax.experimental import pallas as pl


def kernel(queries, keys, values, params):
    raise NotImplementedError("write your pallas kernel here")


# speedup vs baseline: 2.4277x; 2.4277x over previous
import functools

import jax
import jax.numpy as jnp
from jax import lax
from jax.experimental import pallas as pl
from jax.experimental.pallas import tpu as pltpu
from jax.experimental.pallas import tpu_sc as plsc

_D = 256
_H = 4
_HD = 64
_K = 5
_NPROTO = 7
_TN = 2048
_KPAD = 8
_QT = 256

_NEG = -1e30


def _ln_rows(x, g, b):
    m = jnp.mean(x, axis=-1, keepdims=True)
    v = jnp.mean((x - m) ** 2, axis=-1, keepdims=True)
    return (x - m) / jnp.sqrt(v + 1e-5) * g + b


def _rownorm(x):
    return x / (jnp.sqrt(jnp.sum(x * x, axis=-1, keepdims=True)) + 1e-12)


def _qproj_body(q_ref, wa_ref, ba_ref, ga_ref, bga_ref, pr_ref,
                qc_ref, qn_ref, proto_ref):
    x = jnp.dot(q_ref[...], wa_ref[...], preferred_element_type=jnp.float32)
    x = _ln_rows(x + ba_ref[...], ga_ref[...], bga_ref[...])
    x = jnp.maximum(x, 0.0)
    qc_ref[...] = x
    qn = _rownorm(x)
    qn_ref[...] = qn
    pr = pr_ref[...]
    pn = _rownorm(pr)
    pv = lax.dot_general(qn, pn, (((1,), (1,)), ((), ())),
                         preferred_element_type=jnp.float32)
    ci = lax.broadcasted_iota(jnp.int32, pv.shape, 1)
    pv = jnp.where(ci < _NPROTO, pv, _NEG)
    mx = jnp.max(pv, axis=-1, keepdims=True)
    am = jnp.min(jnp.where(pv == mx, ci, _NPROTO), axis=-1, keepdims=True)
    onehot = (ci == am).astype(jnp.float32)
    proto_ref[...] = jnp.dot(onehot, pr, preferred_element_type=jnp.float32)


def _scan_body(keys_ref, wa_ref, ba_ref, ga_ref, bga_ref, qn_ref,
               ov_ref, oi_ref, bv_scr, bi_scr, *, n_real, n_grid, q):
    i = pl.program_id(0)

    @pl.when(i == 0)
    def _init():
        bv_scr[...] = jnp.full((q, _KPAD), _NEG, jnp.float32)
        bi_scr[...] = jnp.zeros((q, _KPAD), jnp.int32)

    x = jnp.dot(keys_ref[...], wa_ref[...], preferred_element_type=jnp.float32)
    x = _ln_rows(x + ba_ref[...], ga_ref[...], bga_ref[...])
    x = jnp.maximum(x, 0.0)
    kn = _rownorm(x)
    sim = lax.dot_general(qn_ref[...], kn, (((1,), (1,)), ((), ())),
                          preferred_element_type=jnp.float32)
    col = lax.broadcasted_iota(jnp.int32, sim.shape, 1)
    gcol = col + i * _TN
    sim = jnp.where(gcol < n_real, sim, _NEG)

    tv, ti = [], []
    work = sim
    for _ in range(_K):
        mx = jnp.max(work, axis=-1, keepdims=True)
        am = jnp.min(jnp.where(work == mx, col, _TN), axis=-1, keepdims=True)
        tv.append(mx)
        ti.append(am + i * _TN)
        work = jnp.where(col == am, _NEG, work)

    pad_v = jnp.full((q, _KPAD - _K), _NEG, jnp.float32)
    pad_i = jnp.zeros((q, _KPAD - _K), jnp.int32)
    cv = jnp.concatenate([bv_scr[...]] + tv + [pad_v], axis=1)
    cidx = jnp.concatenate([bi_scr[...]] + ti + [pad_i], axis=1)
    c16 = lax.broadcasted_iota(jnp.int32, cv.shape, 1)
    nv, ni = [], []
    for _ in range(_K):
        mx = jnp.max(cv, axis=-1, keepdims=True)
        am = jnp.min(jnp.where(cv == mx, c16, 2 * _KPAD), axis=-1,
                     keepdims=True)
        sel = (c16 == am)
        nv.append(mx)
        ni.append(jnp.sum(jnp.where(sel, cidx, 0), axis=-1, keepdims=True))
        cv = jnp.where(sel, _NEG, cv)
    bv_scr[...] = jnp.concatenate(nv + [pad_v], axis=1)
    bi_scr[...] = jnp.concatenate(ni + [pad_i], axis=1)

    @pl.when(i == n_grid - 1)
    def _emit():
        ov_ref[...] = bv_scr[...]
        oi_ref[...] = bi_scr[...]


def _gather_neighbors(keys, values, flat_idx):
    b = flat_idx.shape[0]
    info = plsc.get_sparse_core_info()
    nw = info.num_cores * info.num_subcores
    bpw = b // nw
    mesh = plsc.VectorSubcoreMesh(core_axis_name="c", subcore_axis_name="s")

    @functools.partial(
        pl.kernel, mesh=mesh,
        out_type=(
            jax.ShapeDtypeStruct((b, keys.shape[1]), jnp.float32),
            jax.ShapeDtypeStruct((b, values.shape[1]), jnp.float32),
        ),
        scratch_types=[
            pltpu.VMEM((bpw,), jnp.int32),
            pltpu.VMEM((bpw, keys.shape[1]), jnp.float32),
            pltpu.VMEM((bpw, values.shape[1]), jnp.float32),
            pltpu.SemaphoreType.DMA,
            pltpu.SemaphoreType.DMA,
        ],
    )
    def _gk(keys_hbm, values_hbm, idx_hbm, outk_hbm, outv_hbm,
            idx_v, rk, rv, sem_k, sem_v):
        wid = lax.axis_index("s") * info.num_cores + lax.axis_index("c")
        base = wid * bpw
        pltpu.sync_copy(idx_hbm.at[pl.ds(base, bpw)], idx_v)
        ck = pltpu.async_copy(keys_hbm.at[idx_v], rk, sem_k)
        cvv = pltpu.async_copy(values_hbm.at[idx_v], rv, sem_v)
        ck.wait()
        cvv.wait()
        pltpu.sync_copy(rk, outk_hbm.at[pl.ds(base, bpw)])
        pltpu.sync_copy(rv, outv_hbm.at[pl.ds(base, bpw)])

    return _gk(keys, values, flat_idx)


def _tail_body(qc_ref, proto_ref, kg_ref, vg_ref,
               wa_ref, ba_ref, ga_ref, bga_ref,
               wm_ref, bm_ref, gm_ref, bgm_ref,
               wq_ref, bq_ref, wk_ref, bk_ref, wv_ref, bv_ref,
               wo_ref, bo_ref, g1_ref, b1_ref,
               wf1_ref, bf1_ref, wf2_ref, bf2_ref, g2_ref, b2_ref,
               out_ref):
    qc = qc_ref[...]
    proto = proto_ref[...]
    wk = wk_ref[...]
    bk = bk_ref[...]
    wv = wv_ref[...]
    bv = bv_ref[...]

    qh = jnp.dot(qc, wq_ref[...], preferred_element_type=jnp.float32) + bq_ref[...]
    kh_p = jnp.dot(proto, wk, preferred_element_type=jnp.float32) + bk
    vh_p = jnp.dot(proto, wv, preferred_element_type=jnp.float32) + bv

    kh_n, vh_n = [], []
    for j in range(_K):
        ka = jnp.dot(kg_ref[j], wa_ref[...],
                     preferred_element_type=jnp.float32)
        ka = jnp.maximum(_ln_rows(ka + ba_ref[...], ga_ref[...], bga_ref[...]), 0.0)
        vm = jnp.dot(vg_ref[j], wm_ref[...],
                     preferred_element_type=jnp.float32)
        vm = jnp.maximum(_ln_rows(vm + bm_ref[...], gm_ref[...], bgm_ref[...]), 0.0)
        kh_n.append(jnp.dot(ka, wk, preferred_element_type=jnp.float32) + bk)
        vh_n.append(jnp.dot(vm, wv, preferred_element_type=jnp.float32) + bv)

    ao_cols = []
    for h in range(_H):
        sl = slice(h * _HD, (h + 1) * _HD)
        qh_h = qh[:, sl]
        s_cols = [jnp.sum(qh_h * kh_p[:, sl], axis=-1, keepdims=True)]
        for j in range(_K):
            s_cols.append(jnp.sum(qh_h * kh_n[j][:, sl], axis=-1,
                                  keepdims=True))
        s = jnp.concatenate(s_cols, axis=1) * (1.0 / 8.0)
        s = s - jnp.max(s, axis=-1, keepdims=True)
        e = jnp.exp(s)
        att = e / jnp.sum(e, axis=-1, keepdims=True)
        ao_h = att[:, 0:1] * vh_p[:, sl]
        for j in range(_K):
            ao_h = ao_h + att[:, j + 1:j + 2] * vh_n[j][:, sl]
        ao_cols.append(ao_h)
    ao = jnp.concatenate(ao_cols, axis=1)
    ao = jnp.dot(ao, wo_ref[...], preferred_element_type=jnp.float32) + bo_ref[...]
    out1 = _ln_rows(ao + proto, g1_ref[...], b1_ref[...])
    ffn = jnp.maximum(
        jnp.dot(out1, wf1_ref[...], preferred_element_type=jnp.float32)
        + bf1_ref[...], 0.0)
    ffn = jnp.dot(ffn, wf2_ref[...], preferred_element_type=jnp.float32) + bf2_ref[...]
    out_ref[...] = _ln_rows(ffn + out1, g2_ref[...], b2_ref[...]) + proto


def _row(x):
    return x.reshape(1, -1)


def kernel(queries, keys, values, params):
    p = params
    q = queries.shape[0]
    n = keys.shape[0]
    d_avail = keys.shape[1]
    d_miss = values.shape[1]
    n_grid = -(-n // _TN)

    protos_pad = jnp.concatenate(
        [p['protos'], jnp.zeros((_KPAD - _NPROTO, _D), jnp.float32)], axis=0)

    qc, qn, proto = pl.pallas_call(
        _qproj_body,
        out_shape=(
            jax.ShapeDtypeStruct((q, _D), jnp.float32),
            jax.ShapeDtypeStruct((q, _D), jnp.float32),
            jax.ShapeDtypeStruct((q, _D), jnp.float32),
        ),
    )(queries, p['Wa'], _row(p['ba']), _row(p['ga']), _row(p['bga']),
      protos_pad)

    scan = pl.pallas_call(
        functools.partial(_scan_body, n_real=n, n_grid=n_grid, q=q),
        grid=(n_grid,),
        in_specs=[
            pl.BlockSpec((_TN, d_avail), lambda i: (i, 0)),
            pl.BlockSpec((d_avail, _D), lambda i: (0, 0)),
            pl.BlockSpec((1, _D), lambda i: (0, 0)),
            pl.BlockSpec((1, _D), lambda i: (0, 0)),
            pl.BlockSpec((1, _D), lambda i: (0, 0)),
            pl.BlockSpec((q, _D), lambda i: (0, 0)),
        ],
        out_specs=(
            pl.BlockSpec((q, _KPAD), lambda i: (0, 0)),
            pl.BlockSpec((q, _KPAD), lambda i: (0, 0)),
        ),
        out_shape=(
            jax.ShapeDtypeStruct((q, _KPAD), jnp.float32),
            jax.ShapeDtypeStruct((q, _KPAD), jnp.int32),
        ),
        scratch_shapes=[
            pltpu.VMEM((q, _KPAD), jnp.float32),
            pltpu.VMEM((q, _KPAD), jnp.int32),
        ],
    )(keys, p['Wa'], _row(p['ba']), _row(p['ga']), _row(p['bga']), qn)
    top_idx = scan[1][:, :_K]

    flat_idx = top_idx.T.reshape(-1)
    kg, vg = _gather_neighbors(keys, values, flat_idx)
    kg3 = kg.reshape(_K, q, d_avail)
    vg3 = vg.reshape(_K, q, d_miss)

    full = lambda shp: pl.BlockSpec(shp, lambda t: (0, 0))
    out = pl.pallas_call(
        _tail_body,
        grid=(q // _QT,),
        in_specs=[
            pl.BlockSpec((_QT, _D), lambda t: (t, 0)),
            pl.BlockSpec((_QT, _D), lambda t: (t, 0)),
            pl.BlockSpec((_K, _QT, d_avail), lambda t: (0, t, 0)),
            pl.BlockSpec((_K, _QT, d_miss), lambda t: (0, t, 0)),
            full((d_avail, _D)), full((1, _D)), full((1, _D)), full((1, _D)),
            full((d_miss, _D)), full((1, _D)), full((1, _D)), full((1, _D)),
            full((_D, _D)), full((1, _D)), full((_D, _D)), full((1, _D)),
            full((_D, _D)), full((1, _D)),
            full((_D, _D)), full((1, _D)), full((1, _D)), full((1, _D)),
            full((_D, 4 * _D)), full((1, 4 * _D)),
            full((4 * _D, _D)), full((1, _D)), full((1, _D)), full((1, _D)),
        ],
        out_specs=pl.BlockSpec((_QT, _D), lambda t: (t, 0)),
        out_shape=jax.ShapeDtypeStruct((q, _D), jnp.float32),
    )(qc, proto, kg3, vg3,
      p['Wa'], _row(p['ba']), _row(p['ga']), _row(p['bga']),
      p['Wm'], _row(p['bm']), _row(p['gm']), _row(p['bgm']),
      p['Wq'], _row(p['bq']), p['Wk'], _row(p['bk']),
      p['Wv'], _row(p['bv']),
      p['Wo'], _row(p['bo']), _row(p['g1']), _row(p['b1']),
      p['Wf1'], _row(p['bf1']), p['Wf2'], _row(p['bf2']),
      _row(p['g2']), _row(p['b2']))
    return out
